# baseline (speedup 1.0000x reference)
import functools

import jax
import jax.numpy as jnp
from jax import lax
from jax.experimental import pallas as pl
from jax.experimental.pallas import tpu as pltpu
from jax.experimental.pallas import tpu_sc as plsc

N = 10000
E = 320000
D_IN = 128
H = 32
C = 40

NC = 2
NS = 16
L = 16
NW = NC * NS

NP = 10240
NPT = NP // NS

CHUNK = 256
CPT = 40
NSLOT = 4
EPT_S = CPT * CHUNK
EPAD = NW * EPT_S
EPT_D = E // NW

BLK = 1024
GRID = NP // BLK

_MESH = dict(core_axis_name="c", subcore_axis_name="s")


@functools.partial(
    pl.kernel,
    out_type=jax.ShapeDtypeStruct((NW, NP), jnp.float32),
    mesh=plsc.VectorSubcoreMesh(**_MESH),
    compiler_params=pltpu.CompilerParams(
        use_tc_tiling_on_sc=False, needs_layout_passes=False),
    scratch_types=[
        pltpu.VMEM((EPT_D,), jnp.int32),
        pltpu.VMEM((NP,), jnp.float32),
    ],
)
def _deg_kernel(dst_hbm, out_hbm, didx_v, deg_v):
    c = lax.axis_index("c")
    s = lax.axis_index("s")
    wid = s * NC + c
    zero = jnp.zeros((L,), jnp.float32)

    def zbody(i, carry):
        deg_v[pl.ds(i * L, L)] = zero
        return carry

    lax.fori_loop(0, NP // L, zbody, 0)
    pltpu.sync_copy(dst_hbm.at[pl.ds(wid * EPT_D, EPT_D)], didx_v)
    ones = jnp.ones((L,), jnp.float32)

    def body(i, carry):
        idx = didx_v[pl.ds(i * L, L)]
        plsc.addupdate_scatter(deg_v, [idx], ones)
        return carry

    lax.fori_loop(0, EPT_D // L, body, 0)
    pltpu.sync_copy(deg_v, out_hbm.at[wid])


@functools.partial(
    pl.kernel,
    out_type=jax.ShapeDtypeStruct((NC, NP, H), jnp.float32),
    mesh=plsc.VectorSubcoreMesh(**_MESH),
    compiler_params=pltpu.CompilerParams(use_tc_tiling_on_sc=False),
    scratch_types=(
        [pltpu.VMEM((CPT, CHUNK), jnp.int32),
         pltpu.VMEM((CPT, CHUNK), jnp.int32)]
        + [pltpu.VMEM((CHUNK, H), jnp.float32) for _ in range(NSLOT)]
        + [pltpu.VMEM((NPT, H), jnp.float32)]
        + [pltpu.VMEM_SHARED((NP, H), jnp.float32)]
        + [pltpu.SemaphoreType.DMA for _ in range(2 * NSLOT)]
    ),
)
def _scatter_kernel(hs_hbm, src_hbm, dst_hbm, out_hbm, sidx, didx, *rest):
    rows = rest[:NSLOT]
    zbuf = rest[NSLOT]
    acc_sh = rest[NSLOT + 1]
    gsem = rest[NSLOT + 2:2 * NSLOT + 2]
    ssem = rest[2 * NSLOT + 2:]
    c = lax.axis_index("c")
    s = lax.axis_index("s")
    wid = s * NC + c
    zero = jnp.zeros((L,), jnp.float32)

    def zbody(i, carry):
        zbuf[i, pl.ds(0, L)] = zero
        zbuf[i, pl.ds(L, L)] = zero
        return carry

    lax.fori_loop(0, NPT, zbody, 0)
    pltpu.sync_copy(zbuf, acc_sh.at[pl.ds(s * NPT, NPT)])
    pltpu.sync_copy(src_hbm.at[pl.ds(wid * CPT, CPT)], sidx)
    pltpu.sync_copy(dst_hbm.at[pl.ds(wid * CPT, CPT)], didx)
    plsc.subcore_barrier()

    for b in range(NSLOT):
        pltpu.async_copy(hs_hbm.at[sidx.at[b]], rows[b], gsem[b])

    def chunk(j, carry):
        base = j * NSLOT
        for b in range(NSLOT):
            cc = base + b
            pltpu.make_async_copy(hs_hbm.at[sidx.at[cc]], rows[b],
                                  gsem[b]).wait()
            pltpu.async_copy(rows[b], acc_sh.at[didx.at[cc]], ssem[b],
                             add=True)

        @pl.when(j < CPT // NSLOT - 1)
        def _():
            for b in range(NSLOT):
                cc = base + b
                pltpu.make_async_copy(rows[b], acc_sh.at[didx.at[cc]],
                                      ssem[b]).wait()
                pltpu.async_copy(hs_hbm.at[sidx.at[cc + NSLOT]], rows[b],
                                 gsem[b])

        return carry

    lax.fori_loop(0, CPT // NSLOT, chunk, 0)
    for b in range(NSLOT):
        cc = CPT - NSLOT + b
        pltpu.make_async_copy(rows[b], acc_sh.at[didx.at[cc]], ssem[b]).wait()
    plsc.subcore_barrier()
    pltpu.sync_copy(acc_sh.at[pl.ds(s * NPT, NPT)], zbuf)
    pltpu.sync_copy(zbuf, out_hbm.at[c, pl.ds(s * NPT, NPT)])


def _stage_a_body(x_ref, w_ref, deg_ref, hs_ref, dinv_ref):
    h = jnp.dot(x_ref[...], w_ref[...], preferred_element_type=jnp.float32)
    ones = jnp.ones((NW, 1), jnp.float32)
    deg_col = lax.dot_general(deg_ref[...], ones, (((0,), (0,)), ((), ())),
                              preferred_element_type=jnp.float32)
    dinv = lax.rsqrt(deg_col + 1.0)
    hs_ref[...] = h * dinv
    dinv_ref[...] = jnp.broadcast_to(dinv, (BLK, H))


def _stage_a(xp, W1, degp):
    return pl.pallas_call(
        _stage_a_body,
        grid=(GRID,),
        in_specs=[
            pl.BlockSpec((BLK, D_IN), lambda i: (i, 0)),
            pl.BlockSpec((D_IN, H), lambda i: (0, 0)),
            pl.BlockSpec((NW, BLK), lambda i: (0, i)),
        ],
        out_specs=[
            pl.BlockSpec((BLK, H), lambda i: (i, 0)),
            pl.BlockSpec((BLK, H), lambda i: (i, 0)),
        ],
        out_shape=[
            jax.ShapeDtypeStruct((NP, H), jnp.float32),
            jax.ShapeDtypeStruct((NP, H), jnp.float32),
        ],
    )(xp, W1, degp)


def _stage_bc_body(p_ref, hs_ref, dinv_ref, b_ref, g_ref, bln_ref, w_ref,
                   out_ref):
    dinv = dinv_ref[...]
    e = dinv * (p_ref[0] + p_ref[1] + hs_ref[...]) + b_ref[...]
    r = jnp.maximum(e, 0.0)
    m = jnp.mean(r, axis=1, keepdims=True)
    v = jnp.mean((r - m) ** 2, axis=1, keepdims=True)
    ln = (r - m) / jnp.sqrt(v + 1e-5) * g_ref[...] + bln_ref[...]
    out_ref[...] = jnp.dot(ln, w_ref[...],
                           preferred_element_type=jnp.float32) * dinv


def _stage_bc(parts, hs, dinvb, bias, g, bln, Wn):
    return pl.pallas_call(
        _stage_bc_body,
        grid=(GRID,),
        in_specs=[
            pl.BlockSpec((NC, BLK, H), lambda i: (0, i, 0)),
            pl.BlockSpec((BLK, H), lambda i: (i, 0)),
            pl.BlockSpec((BLK, H), lambda i: (i, 0)),
            pl.BlockSpec((1, H), lambda i: (0, 0)),
            pl.BlockSpec((1, H), lambda i: (0, 0)),
            pl.BlockSpec((1, H), lambda i: (0, 0)),
            pl.BlockSpec((H, H), lambda i: (0, 0)),
        ],
        out_specs=pl.BlockSpec((BLK, H), lambda i: (i, 0)),
        out_shape=jax.ShapeDtypeStruct((NP, H), jnp.float32),
    )(parts, hs, dinvb, bias.reshape(1, H), g.reshape(1, H),
      bln.reshape(1, H), Wn)


def _stage_d_body(p_ref, hs_ref, dinv_ref, b_ref, w1_ref, b1_ref, w2_ref,
                  b2_ref, emb_ref, logp_ref):
    e = dinv_ref[...] * (p_ref[0] + p_ref[1] + hs_ref[...]) + b_ref[...]
    emb_ref[...] = e
    r = jnp.maximum(e, 0.0)
    h1 = jnp.dot(r, w1_ref[...], preferred_element_type=jnp.float32) + b1_ref[...]
    h2 = jnp.dot(h1, w2_ref[...], preferred_element_type=jnp.float32) + b2_ref[...]
    m = jnp.max(h2, axis=1, keepdims=True)
    lse = jnp.log(jnp.sum(jnp.exp(h2 - m), axis=1, keepdims=True)) + m
    logp_ref[...] = h2 - lse


def _stage_d(parts, hs, dinvb, bias, mp1_W, mp1_b, mp2_W, mp2_b):
    return pl.pallas_call(
        _stage_d_body,
        grid=(GRID,),
        in_specs=[
            pl.BlockSpec((NC, BLK, H), lambda i: (0, i, 0)),
            pl.BlockSpec((BLK, H), lambda i: (i, 0)),
            pl.BlockSpec((BLK, H), lambda i: (i, 0)),
            pl.BlockSpec((1, H), lambda i: (0, 0)),
            pl.BlockSpec((H, H), lambda i: (0, 0)),
            pl.BlockSpec((1, H), lambda i: (0, 0)),
            pl.BlockSpec((H, C), lambda i: (0, 0)),
            pl.BlockSpec((1, C), lambda i: (0, 0)),
        ],
        out_specs=[
            pl.BlockSpec((BLK, H), lambda i: (i, 0)),
            pl.BlockSpec((BLK, C), lambda i: (i, 0)),
        ],
        out_shape=[
            jax.ShapeDtypeStruct((NP, H), jnp.float32),
            jax.ShapeDtypeStruct((NP, C), jnp.float32),
        ],
    )(parts, hs, dinvb, bias.reshape(1, H), mp1_W, mp1_b.reshape(1, H),
      mp2_W, mp2_b.reshape(1, C))


def kernel(x, edge_index, W1, b1, W2, b2, W3, b3, ln1_g, ln1_b, ln2_g, ln2_b,
           mp1_W, mp1_b, mp2_W, mp2_b):
    src = edge_index[0]
    dst = edge_index[1]
    xp = jnp.pad(x, ((0, NP - N), (0, 0)))
    fill = jnp.full((EPAD - E,), N, jnp.int32)
    src2 = jnp.concatenate([src, fill]).reshape(EPAD // CHUNK, CHUNK)
    dst2 = jnp.concatenate([dst, fill]).reshape(EPAD // CHUNK, CHUNK)

    degp = _deg_kernel(dst)
    hs1, dinvb = _stage_a(xp, W1, degp)
    p1 = _scatter_kernel(hs1, src2, dst2)
    hs2 = _stage_bc(p1, hs1, dinvb, b1, ln1_g, ln1_b, W2)
    p2 = _scatter_kernel(hs2, src2, dst2)
    hs3 = _stage_bc(p2, hs2, dinvb, b2, ln2_g, ln2_b, W3)
    p3 = _scatter_kernel(hs3, src2, dst2)
    emb, logp = _stage_d(p3, hs3, dinvb, b3, mp1_W, mp1_b, mp2_W, mp2_b)
    return (emb[:N], logp[:N])


# speedup vs baseline: 1.7754x; 1.7754x over previous
import functools

import jax
import jax.numpy as jnp
from jax import lax
from jax.experimental import pallas as pl
from jax.experimental.pallas import tpu as pltpu
from jax.experimental.pallas import tpu_sc as plsc

N = 10000
E = 320000
D_IN = 128
H = 32
C = 40

NC = 2
NS = 16
L = 16
NW = NC * NS

NP = 10240
NPT = NP // NS

CHUNK = 256
CPT = 40
NSLOT = 4
EPT_S = CPT * CHUNK
EPAD = NW * EPT_S
EPT_D = E // NW

BLK = 1024
GRID = NP // BLK

_MESH = dict(core_axis_name="c", subcore_axis_name="s")


@functools.partial(
    pl.kernel,
    out_type=jax.ShapeDtypeStruct((NW, NP), jnp.float32),
    mesh=plsc.VectorSubcoreMesh(**_MESH),
    compiler_params=pltpu.CompilerParams(
        use_tc_tiling_on_sc=False, needs_layout_passes=False),
    scratch_types=[
        pltpu.VMEM((EPT_D,), jnp.int32),
        pltpu.VMEM((NP,), jnp.float32),
    ],
)
def _deg_kernel(dst_hbm, out_hbm, didx_v, deg_v):
    c = lax.axis_index("c")
    s = lax.axis_index("s")
    wid = s * NC + c
    zero = jnp.zeros((L,), jnp.float32)

    def zbody(i, carry):
        deg_v[pl.ds(i * L, L)] = zero
        return carry

    lax.fori_loop(0, NP // L, zbody, 0)
    pltpu.sync_copy(dst_hbm.at[pl.ds(wid * EPT_D, EPT_D)], didx_v)
    ones = jnp.ones((L,), jnp.float32)

    def body(i, carry):
        idx = didx_v[pl.ds(i * L, L)]
        plsc.addupdate_scatter(deg_v, [idx], ones)
        return carry

    lax.fori_loop(0, EPT_D // L, body, 0)
    pltpu.sync_copy(deg_v, out_hbm.at[wid])


@functools.partial(
    pl.kernel,
    out_type=jax.ShapeDtypeStruct((NC, NP, H), jnp.float32),
    mesh=plsc.VectorSubcoreMesh(**_MESH),
    compiler_params=pltpu.CompilerParams(use_tc_tiling_on_sc=False),
    scratch_types=(
        [pltpu.VMEM((CPT, CHUNK), jnp.int32),
         pltpu.VMEM((CPT, CHUNK), jnp.int32)]
        + [pltpu.VMEM((CHUNK, H), jnp.float32) for _ in range(NSLOT)]
        + [pltpu.VMEM((NPT, H), jnp.float32)]
        + [pltpu.VMEM_SHARED((NP, H), jnp.float32),
           pltpu.VMEM_SHARED((NP, H), jnp.float32)]
        + [pltpu.SemaphoreType.DMA for _ in range(2 * NSLOT)]
    ),
)
def _scatter_kernel(hs_hbm, src_hbm, dst_hbm, out_hbm, sidx, didx, *rest):
    rows = rest[:NSLOT]
    zbuf = rest[NSLOT]
    acc_sh = rest[NSLOT + 1]
    hs_sh = rest[NSLOT + 2]
    gsem = rest[NSLOT + 3:2 * NSLOT + 3]
    ssem = rest[2 * NSLOT + 3:]
    c = lax.axis_index("c")
    s = lax.axis_index("s")
    wid = s * NC + c
    zero = jnp.zeros((L,), jnp.float32)

    def zbody(i, carry):
        zbuf[i, pl.ds(0, L)] = zero
        zbuf[i, pl.ds(L, L)] = zero
        return carry

    lax.fori_loop(0, NPT, zbody, 0)
    pltpu.sync_copy(zbuf, acc_sh.at[pl.ds(s * NPT, NPT)])
    pltpu.sync_copy(hs_hbm.at[pl.ds(s * NPT, NPT)],
                    hs_sh.at[pl.ds(s * NPT, NPT)])
    pltpu.sync_copy(src_hbm.at[pl.ds(wid * CPT, CPT)], sidx)
    pltpu.sync_copy(dst_hbm.at[pl.ds(wid * CPT, CPT)], didx)
    plsc.subcore_barrier()

    for b in range(NSLOT):
        pltpu.async_copy(hs_sh.at[sidx.at[b]], rows[b], gsem[b])

    def chunk(j, carry):
        base = j * NSLOT
        for b in range(NSLOT):
            cc = base + b
            pltpu.make_async_copy(hs_sh.at[sidx.at[cc]], rows[b],
                                  gsem[b]).wait()
            pltpu.async_copy(rows[b], acc_sh.at[didx.at[cc]], ssem[b],
                             add=True)

        @pl.when(j < CPT // NSLOT - 1)
        def _():
            for b in range(NSLOT):
                cc = base + b
                pltpu.make_async_copy(rows[b], acc_sh.at[didx.at[cc]],
                                      ssem[b]).wait()
                pltpu.async_copy(hs_sh.at[sidx.at[cc + NSLOT]], rows[b],
                                 gsem[b])

        return carry

    lax.fori_loop(0, CPT // NSLOT, chunk, 0)
    for b in range(NSLOT):
        cc = CPT - NSLOT + b
        pltpu.make_async_copy(rows[b], acc_sh.at[didx.at[cc]], ssem[b]).wait()
    plsc.subcore_barrier()
    pltpu.sync_copy(acc_sh.at[pl.ds(s * NPT, NPT)], zbuf)
    pltpu.sync_copy(zbuf, out_hbm.at[c, pl.ds(s * NPT, NPT)])


def _stage_a_body(x_ref, w_ref, deg_ref, hs_ref, dinv_ref):
    h = jnp.dot(x_ref[...], w_ref[...], preferred_element_type=jnp.float32)
    ones = jnp.ones((NW, 1), jnp.float32)
    deg_col = lax.dot_general(deg_ref[...], ones, (((0,), (0,)), ((), ())),
                              preferred_element_type=jnp.float32)
    dinv = lax.rsqrt(deg_col + 1.0)
    hs_ref[...] = h * dinv
    dinv_ref[...] = jnp.broadcast_to(dinv, (BLK, H))


def _stage_a(xp, W1, degp):
    return pl.pallas_call(
        _stage_a_body,
        grid=(GRID,),
        in_specs=[
            pl.BlockSpec((BLK, D_IN), lambda i: (i, 0)),
            pl.BlockSpec((D_IN, H), lambda i: (0, 0)),
            pl.BlockSpec((NW, BLK), lambda i: (0, i)),
        ],
        out_specs=[
            pl.BlockSpec((BLK, H), lambda i: (i, 0)),
            pl.BlockSpec((BLK, H), lambda i: (i, 0)),
        ],
        out_shape=[
            jax.ShapeDtypeStruct((NP, H), jnp.float32),
            jax.ShapeDtypeStruct((NP, H), jnp.float32),
        ],
    )(xp, W1, degp)


def _stage_bc_body(p_ref, hs_ref, dinv_ref, b_ref, g_ref, bln_ref, w_ref,
                   out_ref):
    dinv = dinv_ref[...]
    e = dinv * (p_ref[0] + p_ref[1] + hs_ref[...]) + b_ref[...]
    r = jnp.maximum(e, 0.0)
    m = jnp.mean(r, axis=1, keepdims=True)
    v = jnp.mean((r - m) ** 2, axis=1, keepdims=True)
    ln = (r - m) / jnp.sqrt(v + 1e-5) * g_ref[...] + bln_ref[...]
    out_ref[...] = jnp.dot(ln, w_ref[...],
                           preferred_element_type=jnp.float32) * dinv


def _stage_bc(parts, hs, dinvb, bias, g, bln, Wn):
    return pl.pallas_call(
        _stage_bc_body,
        grid=(GRID,),
        in_specs=[
            pl.BlockSpec((NC, BLK, H), lambda i: (0, i, 0)),
            pl.BlockSpec((BLK, H), lambda i: (i, 0)),
            pl.BlockSpec((BLK, H), lambda i: (i, 0)),
            pl.BlockSpec((1, H), lambda i: (0, 0)),
            pl.BlockSpec((1, H), lambda i: (0, 0)),
            pl.BlockSpec((1, H), lambda i: (0, 0)),
            pl.BlockSpec((H, H), lambda i: (0, 0)),
        ],
        out_specs=pl.BlockSpec((BLK, H), lambda i: (i, 0)),
        out_shape=jax.ShapeDtypeStruct((NP, H), jnp.float32),
    )(parts, hs, dinvb, bias.reshape(1, H), g.reshape(1, H),
      bln.reshape(1, H), Wn)


def _stage_d_body(p_ref, hs_ref, dinv_ref, b_ref, w1_ref, b1_ref, w2_ref,
                  b2_ref, emb_ref, logp_ref):
    e = dinv_ref[...] * (p_ref[0] + p_ref[1] + hs_ref[...]) + b_ref[...]
    emb_ref[...] = e
    r = jnp.maximum(e, 0.0)
    h1 = jnp.dot(r, w1_ref[...], preferred_element_type=jnp.float32) + b1_ref[...]
    h2 = jnp.dot(h1, w2_ref[...], preferred_element_type=jnp.float32) + b2_ref[...]
    m = jnp.max(h2, axis=1, keepdims=True)
    lse = jnp.log(jnp.sum(jnp.exp(h2 - m), axis=1, keepdims=True)) + m
    logp_ref[...] = h2 - lse


def _stage_d(parts, hs, dinvb, bias, mp1_W, mp1_b, mp2_W, mp2_b):
    return pl.pallas_call(
        _stage_d_body,
        grid=(GRID,),
        in_specs=[
            pl.BlockSpec((NC, BLK, H), lambda i: (0, i, 0)),
            pl.BlockSpec((BLK, H), lambda i: (i, 0)),
            pl.BlockSpec((BLK, H), lambda i: (i, 0)),
            pl.BlockSpec((1, H), lambda i: (0, 0)),
            pl.BlockSpec((H, H), lambda i: (0, 0)),
            pl.BlockSpec((1, H), lambda i: (0, 0)),
            pl.BlockSpec((H, C), lambda i: (0, 0)),
            pl.BlockSpec((1, C), lambda i: (0, 0)),
        ],
        out_specs=[
            pl.BlockSpec((BLK, H), lambda i: (i, 0)),
            pl.BlockSpec((BLK, C), lambda i: (i, 0)),
        ],
        out_shape=[
            jax.ShapeDtypeStruct((NP, H), jnp.float32),
            jax.ShapeDtypeStruct((NP, C), jnp.float32),
        ],
    )(parts, hs, dinvb, bias.reshape(1, H), mp1_W, mp1_b.reshape(1, H),
      mp2_W, mp2_b.reshape(1, C))


def kernel(x, edge_index, W1, b1, W2, b2, W3, b3, ln1_g, ln1_b, ln2_g, ln2_b,
           mp1_W, mp1_b, mp2_W, mp2_b):
    src = edge_index[0]
    dst = edge_index[1]
    xp = jnp.pad(x, ((0, NP - N), (0, 0)))
    fill = jnp.full((EPAD - E,), N, jnp.int32)
    src2 = jnp.concatenate([src, fill]).reshape(EPAD // CHUNK, CHUNK)
    dst2 = jnp.concatenate([dst, fill]).reshape(EPAD // CHUNK, CHUNK)

    degp = _deg_kernel(dst)
    hs1, dinvb = _stage_a(xp, W1, degp)
    p1 = _scatter_kernel(hs1, src2, dst2)
    hs2 = _stage_bc(p1, hs1, dinvb, b1, ln1_g, ln1_b, W2)
    p2 = _scatter_kernel(hs2, src2, dst2)
    hs3 = _stage_bc(p2, hs2, dinvb, b2, ln2_g, ln2_b, W3)
    p3 = _scatter_kernel(hs3, src2, dst2)
    emb, logp = _stage_d(p3, hs3, dinvb, b3, mp1_W, mp1_b, mp2_W, mp2_b)
    return (emb[:N], logp[:N])


# speedup vs baseline: 1.8482x; 1.0410x over previous
import functools

import jax
import jax.numpy as jnp
from jax import lax
from jax.experimental import pallas as pl
from jax.experimental.pallas import tpu as pltpu
from jax.experimental.pallas import tpu_sc as plsc

N = 10000
E = 320000
D_IN = 128
H = 32
C = 40

NC = 2
NS = 16
L = 16
NW = NC * NS

NP = 10240
NPT = NP // NS

CHUNK = 256
EPT = 10240
REAL_EPT = E // NW
CPT = EPT // CHUNK
NSLOT = 4

BLK = 1000
GRID = N // BLK

_MESH = dict(core_axis_name="c", subcore_axis_name="s")


@functools.partial(
    pl.kernel,
    out_type=jax.ShapeDtypeStruct((NW, NP), jnp.float32),
    mesh=plsc.VectorSubcoreMesh(**_MESH),
    compiler_params=pltpu.CompilerParams(
        use_tc_tiling_on_sc=False, needs_layout_passes=False),
    scratch_types=[
        pltpu.VMEM((REAL_EPT,), jnp.int32),
        pltpu.VMEM((NP,), jnp.float32),
    ],
)
def _deg_kernel(ei_hbm, out_hbm, didx_v, deg_v):
    c = lax.axis_index("c")
    s = lax.axis_index("s")
    wid = s * NC + c
    zero = jnp.zeros((L,), jnp.float32)

    def zbody(i, carry):
        deg_v[pl.ds(i * L, L)] = zero
        return carry

    lax.fori_loop(0, NP // L, zbody, 0)
    pltpu.sync_copy(ei_hbm.at[1, pl.ds(wid * REAL_EPT, REAL_EPT)], didx_v)
    ones = jnp.ones((L,), jnp.float32)

    def body(i, carry):
        idx = didx_v[pl.ds(i * L, L)]
        plsc.addupdate_scatter(deg_v, [idx], ones)
        return carry

    lax.fori_loop(0, REAL_EPT // L, body, 0)
    pltpu.sync_copy(deg_v, out_hbm.at[wid])


@functools.partial(
    pl.kernel,
    out_type=jax.ShapeDtypeStruct((NC, NP, H), jnp.float32),
    mesh=plsc.VectorSubcoreMesh(**_MESH),
    compiler_params=pltpu.CompilerParams(use_tc_tiling_on_sc=False),
    scratch_types=(
        [pltpu.VMEM((EPT,), jnp.int32),
         pltpu.VMEM((EPT,), jnp.int32)]
        + [pltpu.VMEM((CHUNK, H), jnp.float32) for _ in range(NSLOT)]
        + [pltpu.VMEM((NPT, H), jnp.float32)]
        + [pltpu.VMEM_SHARED((NP, H), jnp.float32),
           pltpu.VMEM_SHARED((NP, H), jnp.float32)]
        + [pltpu.SemaphoreType.DMA for _ in range(2 * NSLOT)]
    ),
)
def _scatter_kernel(hs_hbm, ei_hbm, out_hbm, sidx, didx, *rest):
    rows = rest[:NSLOT]
    zbuf = rest[NSLOT]
    acc_sh = rest[NSLOT + 1]
    hs_sh = rest[NSLOT + 2]
    gsem = rest[NSLOT + 3:2 * NSLOT + 3]
    ssem = rest[2 * NSLOT + 3:]
    c = lax.axis_index("c")
    s = lax.axis_index("s")
    wid = s * NC + c
    zero = jnp.zeros((L,), jnp.float32)

    def zbody(i, carry):
        zbuf[i, pl.ds(0, L)] = zero
        zbuf[i, pl.ds(L, L)] = zero
        return carry

    lax.fori_loop(0, NPT, zbody, 0)
    pltpu.sync_copy(zbuf, acc_sh.at[pl.ds(s * NPT, NPT)])
    pltpu.sync_copy(hs_hbm.at[pl.ds(s * NPT, NPT)],
                    hs_sh.at[pl.ds(s * NPT, NPT)])
    pltpu.sync_copy(ei_hbm.at[0, pl.ds(wid * REAL_EPT, REAL_EPT)],
                    sidx.at[pl.ds(0, REAL_EPT)])
    pltpu.sync_copy(ei_hbm.at[1, pl.ds(wid * REAL_EPT, REAL_EPT)],
                    didx.at[pl.ds(0, REAL_EPT)])
    padv = jnp.full((L,), N, jnp.int32)

    def fbody(i, carry):
        sidx[pl.ds(REAL_EPT + i * L, L)] = padv
        didx[pl.ds(REAL_EPT + i * L, L)] = padv
        return carry

    lax.fori_loop(0, (EPT - REAL_EPT) // L, fbody, 0)
    plsc.subcore_barrier()

    def soff(cc):
        return sidx.at[pl.ds(cc * CHUNK, CHUNK)]

    def doff(cc):
        return didx.at[pl.ds(cc * CHUNK, CHUNK)]

    for b in range(NSLOT):
        pltpu.async_copy(hs_sh.at[soff(b)], rows[b], gsem[b])

    def chunk(j, carry):
        base = j * NSLOT
        for b in range(NSLOT):
            pltpu.make_async_copy(hs_sh.at[soff(base + b)], rows[b],
                                  gsem[b]).wait()
            pltpu.async_copy(rows[b], acc_sh.at[doff(base + b)], ssem[b],
                             add=True)

        @pl.when(j < CPT // NSLOT - 1)
        def _():
            for b in range(NSLOT):
                pltpu.make_async_copy(rows[b], acc_sh.at[doff(base + b)],
                                      ssem[b]).wait()
                pltpu.async_copy(hs_sh.at[soff(base + b + NSLOT)], rows[b],
                                 gsem[b])

        return carry

    lax.fori_loop(0, CPT // NSLOT, chunk, 0)
    for b in range(NSLOT):
        pltpu.make_async_copy(rows[b], acc_sh.at[doff(CPT - NSLOT + b)],
                              ssem[b]).wait()
    plsc.subcore_barrier()
    pltpu.sync_copy(acc_sh.at[pl.ds(s * NPT, NPT)], zbuf)
    pltpu.sync_copy(zbuf, out_hbm.at[c, pl.ds(s * NPT, NPT)])


BLK_A = 1024
GRID_A = NP // BLK_A


def _stage_a_body(x_ref, w_ref, deg_ref, hs_ref, dinv_ref):
    h = jnp.dot(x_ref[...], w_ref[...], preferred_element_type=jnp.float32)
    ones = jnp.ones((NW, 1), jnp.float32)
    deg_col = lax.dot_general(deg_ref[...], ones, (((0,), (0,)), ((), ())),
                              preferred_element_type=jnp.float32)
    dinv = lax.rsqrt(deg_col + 1.0)
    hs_ref[...] = h * dinv
    dinv_ref[...] = jnp.broadcast_to(dinv, (BLK_A, H))


def _stage_a(xp, W1, degp):
    return pl.pallas_call(
        _stage_a_body,
        grid=(GRID_A,),
        in_specs=[
            pl.BlockSpec((BLK_A, D_IN), lambda i: (i, 0)),
            pl.BlockSpec((D_IN, H), lambda i: (0, 0)),
            pl.BlockSpec((NW, BLK_A), lambda i: (0, i)),
        ],
        out_specs=[
            pl.BlockSpec((BLK_A, H), lambda i: (i, 0)),
            pl.BlockSpec((BLK_A, H), lambda i: (i, 0)),
        ],
        out_shape=[
            jax.ShapeDtypeStruct((NP, H), jnp.float32),
            jax.ShapeDtypeStruct((NP, H), jnp.float32),
        ],
    )(xp, W1, degp)


def _stage_bc_body(p_ref, hs_ref, dinv_ref, b_ref, g_ref, bln_ref, w_ref,
                   out_ref):
    dinv = dinv_ref[...]
    e = dinv * (p_ref[0] + p_ref[1] + hs_ref[...]) + b_ref[...]
    r = jnp.maximum(e, 0.0)
    m = jnp.mean(r, axis=1, keepdims=True)
    v = jnp.mean((r - m) ** 2, axis=1, keepdims=True)
    ln = (r - m) / jnp.sqrt(v + 1e-5) * g_ref[...] + bln_ref[...]
    out_ref[...] = jnp.dot(ln, w_ref[...],
                           preferred_element_type=jnp.float32) * dinv


def _stage_bc(parts, hs, dinvb, bias, g, bln, Wn):
    return pl.pallas_call(
        _stage_bc_body,
        grid=(GRID,),
        in_specs=[
            pl.BlockSpec((NC, BLK, H), lambda i: (0, i, 0)),
            pl.BlockSpec((BLK, H), lambda i: (i, 0)),
            pl.BlockSpec((BLK, H), lambda i: (i, 0)),
            pl.BlockSpec((1, H), lambda i: (0, 0)),
            pl.BlockSpec((1, H), lambda i: (0, 0)),
            pl.BlockSpec((1, H), lambda i: (0, 0)),
            pl.BlockSpec((H, H), lambda i: (0, 0)),
        ],
        out_specs=pl.BlockSpec((BLK, H), lambda i: (i, 0)),
        out_shape=jax.ShapeDtypeStruct((NP, H), jnp.float32),
    )(parts, hs, dinvb, bias.reshape(1, H), g.reshape(1, H),
      bln.reshape(1, H), Wn)


def _stage_d_body(p_ref, hs_ref, dinv_ref, b_ref, w1_ref, b1_ref, w2_ref,
                  b2_ref, emb_ref, logp_ref):
    e = dinv_ref[...] * (p_ref[0] + p_ref[1] + hs_ref[...]) + b_ref[...]
    emb_ref[...] = e
    r = jnp.maximum(e, 0.0)
    h1 = jnp.dot(r, w1_ref[...], preferred_element_type=jnp.float32) + b1_ref[...]
    h2 = jnp.dot(h1, w2_ref[...], preferred_element_type=jnp.float32) + b2_ref[...]
    m = jnp.max(h2, axis=1, keepdims=True)
    lse = jnp.log(jnp.sum(jnp.exp(h2 - m), axis=1, keepdims=True)) + m
    logp_ref[...] = h2 - lse


def _stage_d(parts, hs, dinvb, bias, mp1_W, mp1_b, mp2_W, mp2_b):
    return pl.pallas_call(
        _stage_d_body,
        grid=(GRID,),
        in_specs=[
            pl.BlockSpec((NC, BLK, H), lambda i: (0, i, 0)),
            pl.BlockSpec((BLK, H), lambda i: (i, 0)),
            pl.BlockSpec((BLK, H), lambda i: (i, 0)),
            pl.BlockSpec((1, H), lambda i: (0, 0)),
            pl.BlockSpec((H, H), lambda i: (0, 0)),
            pl.BlockSpec((1, H), lambda i: (0, 0)),
            pl.BlockSpec((H, C), lambda i: (0, 0)),
            pl.BlockSpec((1, C), lambda i: (0, 0)),
        ],
        out_specs=[
            pl.BlockSpec((BLK, H), lambda i: (i, 0)),
            pl.BlockSpec((BLK, C), lambda i: (i, 0)),
        ],
        out_shape=[
            jax.ShapeDtypeStruct((N, H), jnp.float32),
            jax.ShapeDtypeStruct((N, C), jnp.float32),
        ],
    )(parts, hs, dinvb, bias.reshape(1, H), mp1_W, mp1_b.reshape(1, H),
      mp2_W, mp2_b.reshape(1, C))


def kernel(x, edge_index, W1, b1, W2, b2, W3, b3, ln1_g, ln1_b, ln2_g, ln2_b,
           mp1_W, mp1_b, mp2_W, mp2_b):
    degp = _deg_kernel(edge_index)
    xp = jnp.pad(x, ((0, NP - N), (0, 0)))
    hs1, dinvb = _stage_a(xp, W1, degp)
    p1 = _scatter_kernel(hs1, edge_index)
    hs2 = _stage_bc(p1, hs1, dinvb, b1, ln1_g, ln1_b, W2)
    p2 = _scatter_kernel(hs2, edge_index)
    hs3 = _stage_bc(p2, hs2, dinvb, b2, ln2_g, ln2_b, W3)
    p3 = _scatter_kernel(hs3, edge_index)
    emb, logp = _stage_d(p3, hs3, dinvb, b3, mp1_W, mp1_b, mp2_W, mp2_b)
    return (emb, logp)


# speedup vs baseline: 1.9004x; 1.0282x over previous
import functools

import jax
import jax.numpy as jnp
from jax import lax
from jax.experimental import pallas as pl
from jax.experimental.pallas import tpu as pltpu
from jax.experimental.pallas import tpu_sc as plsc

N = 10000
E = 320000
D_IN = 128
H = 32
C = 40

NC = 2
NS = 16
L = 16
NW = NC * NS

NP = 10240
NPT = NP // NS

CHUNK = 256
EPT = 10240
REAL_EPT = E // NW
CPT = EPT // CHUNK
NSLOT = 4

BLK = 2000
GRID = N // BLK

_MESH = dict(core_axis_name="c", subcore_axis_name="s")


@functools.partial(
    pl.kernel,
    out_type=jax.ShapeDtypeStruct((NW, NP), jnp.float32),
    mesh=plsc.VectorSubcoreMesh(**_MESH),
    compiler_params=pltpu.CompilerParams(
        use_tc_tiling_on_sc=False, needs_layout_passes=False),
    scratch_types=[
        pltpu.VMEM((REAL_EPT,), jnp.int32),
        pltpu.VMEM((NP,), jnp.float32),
    ],
)
def _deg_kernel(ei_hbm, out_hbm, didx_v, deg_v):
    c = lax.axis_index("c")
    s = lax.axis_index("s")
    wid = s * NC + c
    zero = jnp.zeros((L,), jnp.float32)

    def zbody(i, carry):
        deg_v[pl.ds(i * L, L)] = zero
        return carry

    lax.fori_loop(0, NP // L, zbody, 0)
    pltpu.sync_copy(ei_hbm.at[1, pl.ds(wid * REAL_EPT, REAL_EPT)], didx_v)
    ones = jnp.ones((L,), jnp.float32)

    def body(i, carry):
        for u in range(4):
            idx = didx_v[pl.ds((i * 4 + u) * L, L)]
            plsc.addupdate_scatter(deg_v, [idx], ones)
        return carry

    lax.fori_loop(0, REAL_EPT // (4 * L), body, 0)
    pltpu.sync_copy(deg_v, out_hbm.at[wid])


@functools.partial(
    pl.kernel,
    out_type=jax.ShapeDtypeStruct((NC, NP, H), jnp.float32),
    mesh=plsc.VectorSubcoreMesh(**_MESH),
    compiler_params=pltpu.CompilerParams(use_tc_tiling_on_sc=False),
    scratch_types=(
        [pltpu.VMEM((EPT,), jnp.int32),
         pltpu.VMEM((EPT,), jnp.int32)]
        + [pltpu.VMEM((CHUNK, H), jnp.float32) for _ in range(NSLOT)]
        + [pltpu.VMEM((NPT, H), jnp.float32)]
        + [pltpu.VMEM_SHARED((NP, H), jnp.float32),
           pltpu.VMEM_SHARED((NP, H), jnp.float32)]
        + [pltpu.SemaphoreType.DMA for _ in range(2 * NSLOT)]
    ),
)
def _scatter_kernel(hs_hbm, ei_hbm, out_hbm, sidx, didx, *rest):
    rows = rest[:NSLOT]
    zbuf = rest[NSLOT]
    acc_sh = rest[NSLOT + 1]
    hs_sh = rest[NSLOT + 2]
    gsem = rest[NSLOT + 3:2 * NSLOT + 3]
    ssem = rest[2 * NSLOT + 3:]
    c = lax.axis_index("c")
    s = lax.axis_index("s")
    wid = s * NC + c
    zero = jnp.zeros((L,), jnp.float32)

    def zbody(i, carry):
        zbuf[i, pl.ds(0, L)] = zero
        zbuf[i, pl.ds(L, L)] = zero
        return carry

    lax.fori_loop(0, NPT, zbody, 0)
    pltpu.sync_copy(zbuf, acc_sh.at[pl.ds(s * NPT, NPT)])
    pltpu.sync_copy(hs_hbm.at[pl.ds(s * NPT, NPT)],
                    hs_sh.at[pl.ds(s * NPT, NPT)])
    pltpu.sync_copy(ei_hbm.at[0, pl.ds(wid * REAL_EPT, REAL_EPT)],
                    sidx.at[pl.ds(0, REAL_EPT)])
    pltpu.sync_copy(ei_hbm.at[1, pl.ds(wid * REAL_EPT, REAL_EPT)],
                    didx.at[pl.ds(0, REAL_EPT)])
    padv = jnp.full((L,), N, jnp.int32)

    def fbody(i, carry):
        sidx[pl.ds(REAL_EPT + i * L, L)] = padv
        didx[pl.ds(REAL_EPT + i * L, L)] = padv
        return carry

    lax.fori_loop(0, (EPT - REAL_EPT) // L, fbody, 0)
    plsc.subcore_barrier()

    def soff(cc):
        return sidx.at[pl.ds(cc * CHUNK, CHUNK)]

    def doff(cc):
        return didx.at[pl.ds(cc * CHUNK, CHUNK)]

    for b in range(NSLOT):
        pltpu.async_copy(hs_sh.at[soff(b)], rows[b], gsem[b])

    def chunk(j, carry):
        base = j * NSLOT
        for b in range(NSLOT):
            pltpu.make_async_copy(hs_sh.at[soff(base + b)], rows[b],
                                  gsem[b]).wait()
            pltpu.async_copy(rows[b], acc_sh.at[doff(base + b)], ssem[b],
                             add=True)

        @pl.when(j < CPT // NSLOT - 1)
        def _():
            for b in range(NSLOT):
                pltpu.make_async_copy(rows[b], acc_sh.at[doff(base + b)],
                                      ssem[b]).wait()
                pltpu.async_copy(hs_sh.at[soff(base + b + NSLOT)], rows[b],
                                 gsem[b])

        return carry

    lax.fori_loop(0, CPT // NSLOT, chunk, 0)
    for b in range(NSLOT):
        pltpu.make_async_copy(rows[b], acc_sh.at[doff(CPT - NSLOT + b)],
                              ssem[b]).wait()
    plsc.subcore_barrier()
    pltpu.sync_copy(acc_sh.at[pl.ds(s * NPT, NPT)], zbuf)
    pltpu.sync_copy(zbuf, out_hbm.at[c, pl.ds(s * NPT, NPT)])


BLK_A = 2048
GRID_A = NP // BLK_A


def _stage_a_body(x_ref, w_ref, deg_ref, hs_ref, dinv_ref):
    h = jnp.dot(x_ref[...], w_ref[...], preferred_element_type=jnp.float32)
    ones = jnp.ones((NW, 1), jnp.float32)
    deg_col = lax.dot_general(deg_ref[...], ones, (((0,), (0,)), ((), ())),
                              preferred_element_type=jnp.float32)
    dinv = lax.rsqrt(deg_col + 1.0)
    hs_ref[...] = h * dinv
    dinv_ref[...] = jnp.broadcast_to(dinv, (BLK_A, H))


def _stage_a(xp, W1, degp):
    return pl.pallas_call(
        _stage_a_body,
        grid=(GRID_A,),
        in_specs=[
            pl.BlockSpec((BLK_A, D_IN), lambda i: (i, 0)),
            pl.BlockSpec((D_IN, H), lambda i: (0, 0)),
            pl.BlockSpec((NW, BLK_A), lambda i: (0, i)),
        ],
        out_specs=[
            pl.BlockSpec((BLK_A, H), lambda i: (i, 0)),
            pl.BlockSpec((BLK_A, H), lambda i: (i, 0)),
        ],
        out_shape=[
            jax.ShapeDtypeStruct((NP, H), jnp.float32),
            jax.ShapeDtypeStruct((NP, H), jnp.float32),
        ],
    )(xp, W1, degp)


def _stage_bc_body(p_ref, hs_ref, dinv_ref, b_ref, g_ref, bln_ref, w_ref,
                   out_ref):
    dinv = dinv_ref[...]
    e = dinv * (p_ref[0] + p_ref[1] + hs_ref[...]) + b_ref[...]
    r = jnp.maximum(e, 0.0)
    m = jnp.mean(r, axis=1, keepdims=True)
    v = jnp.mean((r - m) ** 2, axis=1, keepdims=True)
    ln = (r - m) / jnp.sqrt(v + 1e-5) * g_ref[...] + bln_ref[...]
    out_ref[...] = jnp.dot(ln, w_ref[...],
                           preferred_element_type=jnp.float32) * dinv


def _stage_bc(parts, hs, dinvb, bias, g, bln, Wn):
    return pl.pallas_call(
        _stage_bc_body,
        grid=(GRID,),
        in_specs=[
            pl.BlockSpec((NC, BLK, H), lambda i: (0, i, 0)),
            pl.BlockSpec((BLK, H), lambda i: (i, 0)),
            pl.BlockSpec((BLK, H), lambda i: (i, 0)),
            pl.BlockSpec((1, H), lambda i: (0, 0)),
            pl.BlockSpec((1, H), lambda i: (0, 0)),
            pl.BlockSpec((1, H), lambda i: (0, 0)),
            pl.BlockSpec((H, H), lambda i: (0, 0)),
        ],
        out_specs=pl.BlockSpec((BLK, H), lambda i: (i, 0)),
        out_shape=jax.ShapeDtypeStruct((NP, H), jnp.float32),
    )(parts, hs, dinvb, bias.reshape(1, H), g.reshape(1, H),
      bln.reshape(1, H), Wn)


def _stage_d_body(p_ref, hs_ref, dinv_ref, b_ref, w1_ref, b1_ref, w2_ref,
                  b2_ref, emb_ref, logp_ref):
    e = dinv_ref[...] * (p_ref[0] + p_ref[1] + hs_ref[...]) + b_ref[...]
    emb_ref[...] = e
    r = jnp.maximum(e, 0.0)
    h1 = jnp.dot(r, w1_ref[...], preferred_element_type=jnp.float32) + b1_ref[...]
    h2 = jnp.dot(h1, w2_ref[...], preferred_element_type=jnp.float32) + b2_ref[...]
    m = jnp.max(h2, axis=1, keepdims=True)
    lse = jnp.log(jnp.sum(jnp.exp(h2 - m), axis=1, keepdims=True)) + m
    logp_ref[...] = h2 - lse


def _stage_d(parts, hs, dinvb, bias, mp1_W, mp1_b, mp2_W, mp2_b):
    return pl.pallas_call(
        _stage_d_body,
        grid=(GRID,),
        in_specs=[
            pl.BlockSpec((NC, BLK, H), lambda i: (0, i, 0)),
            pl.BlockSpec((BLK, H), lambda i: (i, 0)),
            pl.BlockSpec((BLK, H), lambda i: (i, 0)),
            pl.BlockSpec((1, H), lambda i: (0, 0)),
            pl.BlockSpec((H, H), lambda i: (0, 0)),
            pl.BlockSpec((1, H), lambda i: (0, 0)),
            pl.BlockSpec((H, C), lambda i: (0, 0)),
            pl.BlockSpec((1, C), lambda i: (0, 0)),
        ],
        out_specs=[
            pl.BlockSpec((BLK, H), lambda i: (i, 0)),
            pl.BlockSpec((BLK, C), lambda i: (i, 0)),
        ],
        out_shape=[
            jax.ShapeDtypeStruct((N, H), jnp.float32),
            jax.ShapeDtypeStruct((N, C), jnp.float32),
        ],
    )(parts, hs, dinvb, bias.reshape(1, H), mp1_W, mp1_b.reshape(1, H),
      mp2_W, mp2_b.reshape(1, C))


def kernel(x, edge_index, W1, b1, W2, b2, W3, b3, ln1_g, ln1_b, ln2_g, ln2_b,
           mp1_W, mp1_b, mp2_W, mp2_b):
    degp = _deg_kernel(edge_index)
    xp = jnp.pad(x, ((0, NP - N), (0, 0)))
    hs1, dinvb = _stage_a(xp, W1, degp)
    p1 = _scatter_kernel(hs1, edge_index)
    hs2 = _stage_bc(p1, hs1, dinvb, b1, ln1_g, ln1_b, W2)
    p2 = _scatter_kernel(hs2, edge_index)
    hs3 = _stage_bc(p2, hs2, dinvb, b2, ln2_g, ln2_b, W3)
    p3 = _scatter_kernel(hs3, edge_index)
    emb, logp = _stage_d(p3, hs3, dinvb, b3, mp1_W, mp1_b, mp2_W, mp2_b)
    return (emb, logp)
